# baseline (speedup 1.0000x reference)
import functools

import jax
import jax.numpy as jnp
from jax import lax
from jax.experimental import pallas as pl
from jax.experimental.pallas import tpu as pltpu

NEG_SLOPE = 0.2
OUT_FEATS = 16
LANE = 128


def _params():
    return pltpu.CompilerParams(
        dimension_semantics=("arbitrary",),
        vmem_limit_bytes=32 * 1024 * 1024,
    )


def _const_spec(shape):
    rank = len(shape)
    return pl.BlockSpec(tuple(shape), lambda i, _r=rank: (0,) * _r)


def _seg_masks(rows, seg):
    t = lax.broadcasted_iota(jnp.int32, (rows, 1), 0) % seg
    return (t != 0).astype(jnp.float32), (t != seg - 1).astype(jnp.float32)


def _cnr3(h_bf, wcat, scale, bias, m_nf, m_nl):
    rows, C = h_bf.shape
    y = jnp.dot(h_bf, wcat, preferred_element_type=jnp.float32)
    z = (y[:, C:2 * C]
         + pltpu.roll(y[:, :C], 1, 0) * m_nf
         + pltpu.roll(y[:, 2 * C:], rows - 1, 0) * m_nl)
    z = z * scale + bias
    return jnp.maximum(z, NEG_SLOPE * z)


def _text_kernel(x_ref, lab_ref, w_ref, s_ref, b_ref, o_ref, scr_ref, *, rep):
    Bb, Ts, C = x_ref.shape
    G, L = w_ref.shape[0], w_ref.shape[1]
    T = Ts * rep
    rt, rows = Bb * Ts, Bb * T

    h0 = x_ref[...].reshape(rt, C).astype(jnp.bfloat16)
    m_nf, m_nl = _seg_masks(rt, Ts)
    nk = C // LANE

    acc = jnp.zeros((rows, C), jnp.float32)
    for g in range(G):
        h = h0
        for l in range(L):
            h = _cnr3(h, w_ref[g, l], s_ref[g, l], b_ref[g, l],
                      m_nf, m_nl).astype(jnp.bfloat16)
        h32 = h.astype(jnp.float32)
        for k in range(nk):
            for j in range(rep):
                scr_ref[g, k, pl.ds(j, rt, rep), :] = \
                    h32[:, LANE * k:LANE * (k + 1)]
        y_up = jnp.concatenate(
            [scr_ref[g, k, pl.ds(0, rows), :] for k in range(nk)], axis=-1)
        acc = acc + y_up * lab_ref[g]
    o_ref[...] = acc.astype(jnp.bfloat16).reshape(Bb, T, C)


def _text_encode(x_text, lab, W, S, Bv, *, rep):
    Bb, Ts, C = x_text.shape
    T = Ts * rep
    G = W.shape[0]
    args = (x_text, lab, W, S, Bv)
    return pl.pallas_call(
        functools.partial(_text_kernel, rep=rep),
        out_shape=jax.ShapeDtypeStruct((Bb, T, C), jnp.bfloat16),
        grid=(1,),
        in_specs=[_const_spec(a.shape) for a in args],
        out_specs=_const_spec((Bb, T, C)),
        scratch_shapes=[pltpu.VMEM((G, C // LANE, Bb * T, LANE), jnp.float32)],
        compiler_params=_params(),
    )(*args)


def _down4s(h_bf, w_pair, scale, bias, scr, seg):
    rows, C = h_bf.shape
    half, to = rows // 2, seg // 2
    nk = C // LANE
    h32 = h_bf.astype(jnp.float32)
    for k in range(nk):
        scr[k, pl.ds(0, rows), :] = h32[:, LANE * k:LANE * (k + 1)]
    he = jnp.concatenate([scr[k, pl.ds(0, half, 2), :] for k in range(nk)],
                         axis=-1).astype(jnp.bfloat16)
    ho = jnp.concatenate([scr[k, pl.ds(1, half, 2), :] for k in range(nk)],
                         axis=-1).astype(jnp.bfloat16)
    ye = jnp.dot(he, w_pair[:, :2 * C], preferred_element_type=jnp.float32)
    yo = jnp.dot(ho, w_pair[:, 2 * C:], preferred_element_type=jnp.float32)
    mf, ml = _seg_masks(half, to)
    z = (ye[:, :C]
         + pltpu.roll(yo[:, :C], 1, 0) * mf
         + yo[:, C:]
         + pltpu.roll(ye[:, C:], half - 1, 0) * ml)
    z = z * scale + bias
    return jnp.maximum(z, NEG_SLOPE * z).astype(jnp.bfloat16)


def _up2s(h_bf, scr):
    ri, C = h_bf.shape
    nk = C // LANE
    h32 = h_bf.astype(jnp.float32)
    for k in range(nk):
        scr[k, pl.ds(0, ri, 2), :] = h32[:, LANE * k:LANE * (k + 1)]
        scr[k, pl.ds(1, ri, 2), :] = h32[:, LANE * k:LANE * (k + 1)]
    return jnp.concatenate([scr[k, pl.ds(0, 2 * ri), :] for k in range(nk)],
                           axis=-1)


def _unet_cls_kernel(x_ref, pw_ref, ps_ref, pb_ref, dw_ref, ds_ref, db_ref,
                     uw_ref, us_ref, ub_ref, cw_ref, cs_ref, cb_ref,
                     chw_ref, chb_ref, ohl_ref, xo_ref, po_ref, ce_ref,
                     scr_ref, *, nc):
    Bb, T, C = x_ref.shape
    rows = Bb * T
    n_lvl = dw_ref.shape[0]
    m_nf, m_nl = _seg_masks(rows, T)

    h_bf = x_ref[...].reshape(rows, C)

    z = None
    for l in range(pw_ref.shape[0]):
        z = _cnr3(h_bf, pw_ref[l], ps_ref[l], pb_ref[l], m_nf, m_nl)
        h_bf = z.astype(jnp.bfloat16)

    residuals = [z]
    seg = T
    for l in range(n_lvl):
        h_bf = _down4s(h_bf, dw_ref[l], ds_ref[l], db_ref[l], scr_ref, seg)
        seg //= 2
        if l < n_lvl - 1:
            residuals.append(h_bf.astype(jnp.float32))

    for l in range(n_lvl):
        hu = _up2s(h_bf, scr_ref)
        seg *= 2
        hsum = hu + residuals[n_lvl - 1 - l]
        mf, ml = _seg_masks(Bb * seg, seg)
        z = _cnr3(hsum.astype(jnp.bfloat16), uw_ref[l], us_ref[l], ub_ref[l],
                  mf, ml)
        h_bf = z.astype(jnp.bfloat16)

    xo_ref[...] = h_bf.reshape(Bb, T, C)

    hc = h_bf
    for l in range(cw_ref.shape[0]):
        hc = _cnr3(hc, cw_ref[l], cs_ref[l], cb_ref[l], m_nf, m_nl
                   ).astype(jnp.bfloat16)
    score = jnp.dot(hc, chw_ref[...], preferred_element_type=jnp.float32)
    score = score + chb_ref[...]
    Fp = score.shape[-1]

    col = lax.broadcasted_iota(jnp.int32, score.shape, 1)
    sm = jnp.where(col < nc, score, -1e30)
    m = jnp.max(sm, axis=-1, keepdims=True)
    e = jnp.where(col < nc, jnp.exp(sm - m), 0.0)
    se = jnp.sum(e, axis=-1, keepdims=True)
    p = e / se
    po_ref[...] = p.reshape(Bb, T, Fp)

    ohl = ohl_ref[...].reshape(rows, Fp)
    sel = jnp.sum(ohl * score, axis=-1, keepdims=True)
    logp_sel = sel - m - jnp.log(se)
    ce = -jnp.sum(logp_sel) / rows
    ce_ref[...] = jnp.full((1, Fp), ce, jnp.float32)


def _unet_cls(x, up, clsp, ohl, *, nc):
    Bb, T, C = x.shape
    Fp = clsp["hw"].shape[-1]
    args = (x, up["pw"], up["ps"], up["pb"], up["dw"], up["ds"], up["db"],
            up["uw"], up["us"], up["ub"],
            clsp["w"], clsp["s"], clsp["b"], clsp["hw"], clsp["hb"], ohl)
    return pl.pallas_call(
        functools.partial(_unet_cls_kernel, nc=nc),
        out_shape=(jax.ShapeDtypeStruct((Bb, T, C), jnp.bfloat16),
                   jax.ShapeDtypeStruct((Bb, T, Fp), jnp.float32),
                   jax.ShapeDtypeStruct((1, Fp), jnp.float32)),
        grid=(1,),
        in_specs=[_const_spec(a.shape) for a in args],
        out_specs=(_const_spec((Bb, T, C)), _const_spec((Bb, T, Fp)),
                   _const_spec((1, Fp))),
        scratch_shapes=[pltpu.VMEM((C // LANE, Bb * T, LANE), jnp.float32)],
        compiler_params=_params(),
    )(*args)


def _dec_kernel(x_ref, p_ref, w_ref, s_ref, b_ref, hw_ref, hb_ref, o_ref):
    Bb, T, C = x_ref.shape
    G, L = w_ref.shape[0], w_ref.shape[1]
    Fp = o_ref.shape[-1]
    rows = Bb * T
    m_nf, m_nl = _seg_masks(rows, T)

    x_bf = x_ref[...].reshape(rows, C)
    p = p_ref[...].reshape(rows, p_ref.shape[-1])
    colp = lax.broadcasted_iota(jnp.int32, p.shape, 1)

    acc = jnp.zeros((rows, Fp), jnp.float32)
    for g in range(G):
        h = x_bf
        for l in range(L):
            h = _cnr3(h, w_ref[g, l], s_ref[g, l], b_ref[g, l],
                      m_nf, m_nl).astype(jnp.bfloat16)
        y = jnp.dot(h, hw_ref[g], preferred_element_type=jnp.float32) + hb_ref[g]
        pg = jnp.sum(jnp.where(colp == g, p, 0.0), axis=-1, keepdims=True)
        acc = acc + y * pg
    o_ref[...] = acc.reshape(Bb, T, Fp)


def _decode(x, p, decp):
    Bb, T, C = x.shape
    Fp = decp["hw"].shape[-1]
    args = (x, p, decp["w"], decp["s"], decp["b"], decp["hw"], decp["hb"])
    return pl.pallas_call(
        _dec_kernel,
        out_shape=jax.ShapeDtypeStruct((Bb, T, Fp), jnp.float32),
        grid=(1,),
        in_specs=[_const_spec(a.shape) for a in args],
        out_specs=_const_spec((Bb, T, Fp)),
        compiler_params=_params(),
    )(*args)


def kernel(text_W, text_S, text_B,
           unet_pw, unet_ps, unet_pb, unet_dw, unet_ds, unet_db,
           unet_uw, unet_us, unet_ub,
           cls_w, cls_s, cls_b, cls_hw, cls_hb,
           dec_w, dec_s, dec_b, dec_hw, dec_hb,
           text, labels, labels_pos):
    Bb, Ts, text_ch = text.shape
    T = labels.shape[1]
    rep = T // Ts
    ncp = text_W.shape[0]
    C = text_W.shape[2]
    nc = dec_w.shape[0]

    x_text = jnp.pad(text, ((0, 0), (0, 0), (0, C - text_ch)))
    lab = jax.nn.one_hot(labels_pos, ncp, dtype=jnp.float32)
    lab = jnp.transpose(lab, (2, 0, 1)).reshape(ncp, Bb * T, 1)
    lab = jnp.broadcast_to(lab, (ncp, Bb * T, C)).astype(jnp.bfloat16)
    Fp = dec_hw.shape[-1]
    ohl = jax.nn.one_hot(labels, nc, dtype=jnp.float32)
    ohl = jnp.pad(ohl, ((0, 0), (0, 0), (0, Fp - nc))).astype(jnp.bfloat16)
    unet_dw = jnp.concatenate(
        [unet_dw[:, :, C:2 * C], unet_dw[:, :, 3 * C:],
         unet_dw[:, :, :C], unet_dw[:, :, 2 * C:3 * C]], axis=-1)

    x = _text_encode(x_text, lab, text_W, text_S, text_B, rep=rep)

    up = dict(pw=unet_pw, ps=unet_ps, pb=unet_pb,
              dw=unet_dw, ds=unet_ds, db=unet_db,
              uw=unet_uw, us=unet_us, ub=unet_ub)
    clsp = dict(w=cls_w, s=cls_s, b=cls_b, hw=cls_hw, hb=cls_hb)
    x_unet, p_pad, ce_out = _unet_cls(x, up, clsp, ohl, nc=nc)

    out_pad = _decode(x_unet, p_pad,
                      dict(w=dec_w, s=dec_s, b=dec_b, hw=dec_hw, hb=dec_hb))

    return out_pad[:, :, :OUT_FEATS], [ce_out[0, 0]]


# speedup vs baseline: 1.0132x; 1.0132x over previous
import functools

import jax
import jax.numpy as jnp
from jax import lax
from jax.experimental import pallas as pl
from jax.experimental.pallas import tpu as pltpu

NEG_SLOPE = 0.2
OUT_FEATS = 16
LANE = 128


def _params():
    return pltpu.CompilerParams(
        dimension_semantics=("arbitrary",),
        vmem_limit_bytes=32 * 1024 * 1024,
    )


def _const_spec(shape):
    rank = len(shape)
    return pl.BlockSpec(tuple(shape), lambda i, _r=rank: (0,) * _r)


def _seg_masks(rows, seg):
    t = lax.broadcasted_iota(jnp.int32, (rows, 1), 0) % seg
    return (t != 0).astype(jnp.float32), (t != seg - 1).astype(jnp.float32)


def _cnr3(h_bf, wcat, scale, bias, m_nf, m_nl):
    rows, C = h_bf.shape
    y = jnp.dot(h_bf, wcat, preferred_element_type=jnp.float32)
    z = (y[:, C:2 * C]
         + pltpu.roll(y[:, :C], 1, 0) * m_nf
         + pltpu.roll(y[:, 2 * C:], rows - 1, 0) * m_nl)
    z = z * scale + bias
    return jnp.maximum(z, NEG_SLOPE * z)


def _text_kernel(x_ref, lab_ref, w_ref, s_ref, b_ref, o_ref, scr_ref, *, rep):
    Bb, Ts, C = x_ref.shape
    G, L = w_ref.shape[0], w_ref.shape[1]
    T = Ts * rep
    rt, rows = Bb * Ts, Bb * T

    h0 = x_ref[...].reshape(rt, C).astype(jnp.bfloat16)
    m_nf, m_nl = _seg_masks(rt, Ts)
    nk = C // LANE

    acc = jnp.zeros((rows, C), jnp.float32)
    for g in range(G):
        h = h0
        for l in range(L):
            h = _cnr3(h, w_ref[g, l], s_ref[g, l], b_ref[g, l],
                      m_nf, m_nl).astype(jnp.bfloat16)
        h32 = h.astype(jnp.float32)
        for k in range(nk):
            for j in range(rep):
                scr_ref[g, k, pl.ds(j, rt, rep), :] = \
                    h32[:, LANE * k:LANE * (k + 1)]
        y_up = jnp.concatenate(
            [scr_ref[g, k, pl.ds(0, rows), :] for k in range(nk)], axis=-1)
        acc = acc + y_up * lab_ref[g]
    o_ref[...] = acc.reshape(Bb, T, C)


def _text_encode(x_text, lab, W, S, Bv, *, rep):
    Bb, Ts, C = x_text.shape
    T = Ts * rep
    G = W.shape[0]
    args = (x_text, lab, W, S, Bv)
    return pl.pallas_call(
        functools.partial(_text_kernel, rep=rep),
        out_shape=jax.ShapeDtypeStruct((Bb, T, C), jnp.float32),
        grid=(1,),
        in_specs=[_const_spec(a.shape) for a in args],
        out_specs=_const_spec((Bb, T, C)),
        scratch_shapes=[pltpu.VMEM((G, C // LANE, Bb * T, LANE), jnp.float32)],
        compiler_params=_params(),
    )(*args)


def _down4s(h_bf, w_pair, scale, bias, scr, seg):
    rows, C = h_bf.shape
    half, to = rows // 2, seg // 2
    nk = C // LANE
    h32 = h_bf.astype(jnp.float32)
    for k in range(nk):
        scr[k, pl.ds(0, rows), :] = h32[:, LANE * k:LANE * (k + 1)]
    he = jnp.concatenate([scr[k, pl.ds(0, half, 2), :] for k in range(nk)],
                         axis=-1).astype(jnp.bfloat16)
    ho = jnp.concatenate([scr[k, pl.ds(1, half, 2), :] for k in range(nk)],
                         axis=-1).astype(jnp.bfloat16)
    ye = jnp.dot(he, w_pair[:, :2 * C], preferred_element_type=jnp.float32)
    yo = jnp.dot(ho, w_pair[:, 2 * C:], preferred_element_type=jnp.float32)
    mf, ml = _seg_masks(half, to)
    z = (ye[:, :C]
         + pltpu.roll(yo[:, :C], 1, 0) * mf
         + yo[:, C:]
         + pltpu.roll(ye[:, C:], half - 1, 0) * ml)
    z = z * scale + bias
    return jnp.maximum(z, NEG_SLOPE * z).astype(jnp.bfloat16)


def _up2s(h_bf, scr):
    ri, C = h_bf.shape
    nk = C // LANE
    h32 = h_bf.astype(jnp.float32)
    for k in range(nk):
        scr[k, pl.ds(0, ri, 2), :] = h32[:, LANE * k:LANE * (k + 1)]
        scr[k, pl.ds(1, ri, 2), :] = h32[:, LANE * k:LANE * (k + 1)]
    return jnp.concatenate([scr[k, pl.ds(0, 2 * ri), :] for k in range(nk)],
                           axis=-1)


def _unet_cls_kernel(x_ref, pw_ref, ps_ref, pb_ref, dw_ref, ds_ref, db_ref,
                     uw_ref, us_ref, ub_ref, cw_ref, cs_ref, cb_ref,
                     chw_ref, chb_ref, ohl_ref, xo_ref, po_ref, ce_ref,
                     scr_ref, *, nc):
    Bb, T, C = x_ref.shape
    rows = Bb * T
    n_lvl = dw_ref.shape[0]
    m_nf, m_nl = _seg_masks(rows, T)

    h_bf = x_ref[...].reshape(rows, C).astype(jnp.bfloat16)

    z = None
    for l in range(pw_ref.shape[0]):
        z = _cnr3(h_bf, pw_ref[l], ps_ref[l], pb_ref[l], m_nf, m_nl)
        h_bf = z.astype(jnp.bfloat16)

    residuals = [z]
    seg = T
    for l in range(n_lvl):
        h_bf = _down4s(h_bf, dw_ref[l], ds_ref[l], db_ref[l], scr_ref, seg)
        seg //= 2
        if l < n_lvl - 1:
            residuals.append(h_bf.astype(jnp.float32))

    for l in range(n_lvl):
        hu = _up2s(h_bf, scr_ref)
        seg *= 2
        hsum = hu + residuals[n_lvl - 1 - l]
        mf, ml = _seg_masks(Bb * seg, seg)
        z = _cnr3(hsum.astype(jnp.bfloat16), uw_ref[l], us_ref[l], ub_ref[l],
                  mf, ml)
        h_bf = z.astype(jnp.bfloat16)

    xo_ref[...] = z.reshape(Bb, T, C)

    hc = h_bf
    for l in range(cw_ref.shape[0]):
        hc = _cnr3(hc, cw_ref[l], cs_ref[l], cb_ref[l], m_nf, m_nl
                   ).astype(jnp.bfloat16)
    score = jnp.dot(hc, chw_ref[...], preferred_element_type=jnp.float32)
    score = score + chb_ref[...]
    Fp = score.shape[-1]

    col = lax.broadcasted_iota(jnp.int32, score.shape, 1)
    sm = jnp.where(col < nc, score, -1e30)
    m = jnp.max(sm, axis=-1, keepdims=True)
    e = jnp.where(col < nc, jnp.exp(sm - m), 0.0)
    se = jnp.sum(e, axis=-1, keepdims=True)
    p = e / se
    po_ref[...] = p.reshape(Bb, T, Fp)

    ohl = ohl_ref[...].reshape(rows, Fp)
    sel = jnp.sum(ohl * score, axis=-1, keepdims=True)
    logp_sel = sel - m - jnp.log(se)
    ce = -jnp.sum(logp_sel) / rows
    ce_ref[...] = jnp.full((1, Fp), ce, jnp.float32)


def _unet_cls(x, up, clsp, ohl, *, nc):
    Bb, T, C = x.shape
    Fp = clsp["hw"].shape[-1]
    args = (x, up["pw"], up["ps"], up["pb"], up["dw"], up["ds"], up["db"],
            up["uw"], up["us"], up["ub"],
            clsp["w"], clsp["s"], clsp["b"], clsp["hw"], clsp["hb"], ohl)
    return pl.pallas_call(
        functools.partial(_unet_cls_kernel, nc=nc),
        out_shape=(jax.ShapeDtypeStruct((Bb, T, C), jnp.float32),
                   jax.ShapeDtypeStruct((Bb, T, Fp), jnp.float32),
                   jax.ShapeDtypeStruct((1, Fp), jnp.float32)),
        grid=(1,),
        in_specs=[_const_spec(a.shape) for a in args],
        out_specs=(_const_spec((Bb, T, C)), _const_spec((Bb, T, Fp)),
                   _const_spec((1, Fp))),
        scratch_shapes=[pltpu.VMEM((C // LANE, Bb * T, LANE), jnp.float32)],
        compiler_params=_params(),
    )(*args)


def _dec_kernel(x_ref, p_ref, w_ref, s_ref, b_ref, hw_ref, hb_ref, o_ref):
    Bb, T, C = x_ref.shape
    G, L = w_ref.shape[0], w_ref.shape[1]
    Fp = o_ref.shape[-1]
    rows = Bb * T
    m_nf, m_nl = _seg_masks(rows, T)

    x_bf = x_ref[...].reshape(rows, C).astype(jnp.bfloat16)
    p = p_ref[...].reshape(rows, p_ref.shape[-1])
    colp = lax.broadcasted_iota(jnp.int32, p.shape, 1)

    acc = jnp.zeros((rows, Fp), jnp.float32)
    for g in range(G):
        h = x_bf
        for l in range(L):
            h = _cnr3(h, w_ref[g, l], s_ref[g, l], b_ref[g, l],
                      m_nf, m_nl).astype(jnp.bfloat16)
        y = jnp.dot(h, hw_ref[g], preferred_element_type=jnp.float32) + hb_ref[g]
        pg = jnp.sum(jnp.where(colp == g, p, 0.0), axis=-1, keepdims=True)
        acc = acc + y * pg
    o_ref[...] = acc.reshape(Bb, T, Fp)


def _decode(x, p, decp):
    Bb, T, C = x.shape
    Fp = decp["hw"].shape[-1]
    args = (x, p, decp["w"], decp["s"], decp["b"], decp["hw"], decp["hb"])
    return pl.pallas_call(
        _dec_kernel,
        out_shape=jax.ShapeDtypeStruct((Bb, T, Fp), jnp.float32),
        grid=(1,),
        in_specs=[_const_spec(a.shape) for a in args],
        out_specs=_const_spec((Bb, T, Fp)),
        compiler_params=_params(),
    )(*args)


def kernel(text_W, text_S, text_B,
           unet_pw, unet_ps, unet_pb, unet_dw, unet_ds, unet_db,
           unet_uw, unet_us, unet_ub,
           cls_w, cls_s, cls_b, cls_hw, cls_hb,
           dec_w, dec_s, dec_b, dec_hw, dec_hb,
           text, labels, labels_pos):
    Bb, Ts, text_ch = text.shape
    T = labels.shape[1]
    rep = T // Ts
    ncp = text_W.shape[0]
    C = text_W.shape[2]
    nc = dec_w.shape[0]

    x_text = jnp.pad(text, ((0, 0), (0, 0), (0, C - text_ch)))
    lab = jax.nn.one_hot(labels_pos, ncp, dtype=jnp.float32)
    lab = jnp.transpose(lab, (2, 0, 1)).reshape(ncp, Bb * T, 1)
    lab = jnp.broadcast_to(lab, (ncp, Bb * T, C)).astype(jnp.bfloat16)
    Fp = dec_hw.shape[-1]
    ohl = jax.nn.one_hot(labels, nc, dtype=jnp.float32)
    ohl = jnp.pad(ohl, ((0, 0), (0, 0), (0, Fp - nc))).astype(jnp.bfloat16)
    unet_dw = jnp.concatenate(
        [unet_dw[:, :, C:2 * C], unet_dw[:, :, 3 * C:],
         unet_dw[:, :, :C], unet_dw[:, :, 2 * C:3 * C]], axis=-1)

    x = _text_encode(x_text, lab, text_W, text_S, text_B, rep=rep)

    up = dict(pw=unet_pw, ps=unet_ps, pb=unet_pb,
              dw=unet_dw, ds=unet_ds, db=unet_db,
              uw=unet_uw, us=unet_us, ub=unet_ub)
    clsp = dict(w=cls_w, s=cls_s, b=cls_b, hw=cls_hw, hb=cls_hb)
    x_unet, p_pad, ce_out = _unet_cls(x, up, clsp, ohl, nc=nc)

    out_pad = _decode(x_unet, p_pad,
                      dict(w=dec_w, s=dec_s, b=dec_b, hw=dec_hw, hb=dec_hb))

    return out_pad[:, :, :OUT_FEATS], [ce_out[0, 0]]
